# baseline (speedup 1.0000x reference)
import jax
import jax.numpy as jnp
from jax import lax
from jax.experimental import pallas as pl
from jax.experimental.pallas import tpu as pltpu

HID = 128
OUT = 32
DICT = 100000
FDIM = 6
RDIM = 1
IN_DIM = FDIM + RDIM + OUT + 1
NG = 5
EPS = 1e-8

BLK = 10000
NBLK = 10
MINIT = -3e38


def _body(xt_ref, h_ref, c_ref, wi_ref, bi_ref, wh_ref, bh_ref,
          fkT_ref, fv_ref, rkT_ref, rv_ref,
          ws_ref, bs_ref, wa_ref, ba_ref, wv_ref, bv_ref,
          ht_out, ct_out, pi_out, v_out,
          st_ref, accf_ref, accr_ref):
    i = pl.program_id(0)

    @pl.when(i == 0)
    def _init():
        st_ref[0] = MINIT
        st_ref[1] = 0.0
        st_ref[2] = MINIT
        st_ref[3] = 0.0
        accf_ref[...] = jnp.zeros_like(accf_ref)
        accr_ref[...] = jnp.zeros_like(accr_ref)

    xt = xt_ref[...]
    qf = xt[:, :FDIM]
    qr = xt[:, FDIM:FDIM + RDIM]
    qnsq_f = jnp.sum(qf * qf) + EPS
    qnsq_r = jnp.sum(qr * qr) + EPS

    fkT = fkT_ref[0].T
    ksq_f = jnp.sum(fkT * fkT, axis=0, keepdims=True) + EPS
    dot_f = lax.dot_general(qf, fkT, (((1,), (0,)), ((), ())),
                            preferred_element_type=jnp.float32)
    sims_f = dot_f / (jnp.sqrt(ksq_f) * jnp.sqrt(qnsq_f))

    m_old = st_ref[0]
    s_old = st_ref[1]
    m_new = jnp.maximum(m_old, jnp.max(sims_f))
    corr = jnp.exp(m_old - m_new)
    ew = jnp.exp(sims_f - m_new)
    st_ref[0] = m_new
    st_ref[1] = s_old * corr + jnp.sum(ew)
    pav = lax.dot_general(ew, fv_ref[...], (((1,), (0,)), ((), ())),
                          preferred_element_type=jnp.float32)
    accf_ref[...] = accf_ref[...] * corr + pav

    rkT = rkT_ref[0]
    ksq_r = rkT * rkT + EPS
    sims_r = (rkT * qr) / (jnp.sqrt(ksq_r) * jnp.sqrt(qnsq_r))

    m_old_r = st_ref[2]
    s_old_r = st_ref[3]
    m_new_r = jnp.maximum(m_old_r, jnp.max(sims_r))
    corr_r = jnp.exp(m_old_r - m_new_r)
    ew_r = jnp.exp(sims_r - m_new_r)
    st_ref[2] = m_new_r
    st_ref[3] = s_old_r * corr_r + jnp.sum(ew_r)
    pav_r = lax.dot_general(ew_r, rv_ref[...], (((1,), (0,)), ((), ())),
                            preferred_element_type=jnp.float32)
    accr_ref[...] = accr_ref[...] * corr_r + pav_r

    @pl.when(i == NBLK - 1)
    def _finish():
        memfun = jnp.tanh(accf_ref[...] / st_ref[1])
        memrule = jnp.tanh(accr_ref[...] / st_ref[3])

        h2 = h_ref[...]
        c2 = c_ref[...]
        preact = (lax.dot_general(xt, wi_ref[...], (((1,), (0,)), ((), ())),
                                  preferred_element_type=jnp.float32)
                  + bi_ref[...]
                  + lax.dot_general(h2, wh_ref[...], (((1,), (0,)), ((), ())),
                                    preferred_element_type=jnp.float32)
                  + bh_ref[...])
        g = jax.nn.sigmoid(preact[:, :NG * HID])
        f_t = g[:, :HID]
        i_t = g[:, HID:2 * HID]
        o_t = g[:, 2 * HID:3 * HID]
        fun_t = g[:, 3 * HID:4 * HID]
        rul_t = g[:, 4 * HID:5 * HID]
        c_new = jnp.tanh(preact[:, NG * HID:])
        c_t = f_t * c2 + i_t * c_new + fun_t * memfun + rul_t * memrule
        h_t = o_t * jnp.tanh(c_t)

        sh = jnp.maximum(
            lax.dot_general(h_t, ws_ref[...], (((1,), (0,)), ((), ())),
                            preferred_element_type=jnp.float32) + bs_ref[...],
            0.0)
        logits = lax.dot_general(sh, wa_ref[...], (((1,), (0,)), ((), ())),
                                 preferred_element_type=jnp.float32) + ba_ref[...]
        lmax = jnp.max(logits)
        el = jnp.exp(logits - lmax)
        pi = el / jnp.sum(el)
        v = (jnp.sum(sh * wv_ref[...], axis=1, keepdims=True)
             + bv_ref[...])

        ht_out[...] = h_t
        ct_out[...] = c_t
        pi_out[...] = pi
        v_out[...] = v


def kernel(x_t, h, c, W_i2h, b_i2h, W_h2h, b_h2h, f_keys, f_vals,
           r_keys, r_vals, W_s, b_s, W_a, b_a, W_v, b_v):
    xt = x_t.reshape(1, IN_DIM)
    h2 = h.reshape(1, HID)
    c2 = c.reshape(1, HID)
    fk3 = f_keys.reshape(NBLK, BLK, FDIM)
    rk3 = r_keys.reshape(NBLK, 1, BLK)
    bi = b_i2h.reshape(1, -1)
    bh = b_h2h.reshape(1, -1)
    bs = b_s.reshape(1, -1)
    ba = b_a.reshape(1, -1)
    bv = b_v.reshape(1, 1)
    wvT = W_v.reshape(1, HID)

    cidx = lambda i: (0, 0)
    h_t, c_t, pi, v = pl.pallas_call(
        _body,
        grid=(NBLK,),
        in_specs=[
            pl.BlockSpec((1, IN_DIM), cidx),
            pl.BlockSpec((1, HID), cidx),
            pl.BlockSpec((1, HID), cidx),
            pl.BlockSpec((IN_DIM, (NG + 1) * HID), cidx),
            pl.BlockSpec((1, (NG + 1) * HID), cidx),
            pl.BlockSpec((HID, (NG + 1) * HID), cidx),
            pl.BlockSpec((1, (NG + 1) * HID), cidx),
            pl.BlockSpec((1, BLK, FDIM), lambda i: (i, 0, 0)),
            pl.BlockSpec((BLK, HID), lambda i: (i, 0)),
            pl.BlockSpec((1, 1, BLK), lambda i: (i, 0, 0)),
            pl.BlockSpec((BLK, HID), lambda i: (i, 0)),
            pl.BlockSpec((HID, HID), cidx),
            pl.BlockSpec((1, HID), cidx),
            pl.BlockSpec((HID, OUT), cidx),
            pl.BlockSpec((1, OUT), cidx),
            pl.BlockSpec((1, HID), cidx),
            pl.BlockSpec((1, 1), cidx),
        ],
        out_specs=[
            pl.BlockSpec((1, HID), cidx),
            pl.BlockSpec((1, HID), cidx),
            pl.BlockSpec((1, OUT), cidx),
            pl.BlockSpec((1, 1), cidx),
        ],
        out_shape=[
            jax.ShapeDtypeStruct((1, HID), jnp.float32),
            jax.ShapeDtypeStruct((1, HID), jnp.float32),
            jax.ShapeDtypeStruct((1, OUT), jnp.float32),
            jax.ShapeDtypeStruct((1, 1), jnp.float32),
        ],
        scratch_shapes=[
            pltpu.SMEM((4,), jnp.float32),
            pltpu.VMEM((1, HID), jnp.float32),
            pltpu.VMEM((1, HID), jnp.float32),
        ],
        compiler_params=pltpu.CompilerParams(
            dimension_semantics=("arbitrary",)),
    )(xt, h2, c2, W_i2h, bi, W_h2h, bh, fk3, f_vals, rk3, r_vals,
      W_s, bs, W_a, ba, wvT, bv)

    a_t = jax.random.categorical(jax.random.key(1), jnp.log(pi + EPS), axis=-1)
    prob_a_t = jnp.log(pi[0, a_t[0]] + EPS)
    h_out = h_t.reshape(1, 1, HID)
    c_out = c_t.reshape(1, 1, HID)
    return (a_t, prob_a_t, v, h_out, c_out, pi)


# speedup vs baseline: 1.7252x; 1.7252x over previous
import jax
import jax.numpy as jnp
from jax import lax
from jax.experimental import pallas as pl
from jax.experimental.pallas import tpu as pltpu

HID = 128
OUT = 32
DICT = 100000
FDIM = 6
RDIM = 1
IN_DIM = FDIM + RDIM + OUT + 1
NG = 5
EPS = 1e-8

BLK = 10000
NBLK = 10
MINIT = -3e38


def _body(xt_ref, h_ref, c_ref, wi_ref, bi_ref, wh_ref, bh_ref,
          fkT_ref, fv_ref, rkT_ref, rv_ref,
          ws_ref, bs_ref, wa_ref, ba_ref, wv_ref, bv_ref,
          ht_out, ct_out, pi_out, v_out,
          st_ref, accf_ref, accr_ref):
    i = pl.program_id(0)

    @pl.when(i == 0)
    def _init():
        st_ref[0] = MINIT
        st_ref[1] = 0.0
        st_ref[2] = MINIT
        st_ref[3] = 0.0
        accf_ref[...] = jnp.zeros_like(accf_ref)
        accr_ref[...] = jnp.zeros_like(accr_ref)

    xt = xt_ref[...]
    qf = xt[:, :FDIM]
    qr = xt[:, FDIM:FDIM + RDIM]
    qnsq_f = jnp.sum(qf * qf) + EPS
    qnsq_r = jnp.sum(qr * qr) + EPS

    fkT = fkT_ref[0]
    ksq_f = jnp.sum(fkT * fkT, axis=0, keepdims=True) + EPS
    dot_f = lax.dot_general(qf, fkT, (((1,), (0,)), ((), ())),
                            preferred_element_type=jnp.float32)
    sims_f = dot_f / (jnp.sqrt(ksq_f) * jnp.sqrt(qnsq_f))

    m_old = st_ref[0]
    s_old = st_ref[1]
    m_new = jnp.maximum(m_old, jnp.max(sims_f))
    corr = jnp.exp(m_old - m_new)
    ew = jnp.exp(sims_f - m_new)
    st_ref[0] = m_new
    st_ref[1] = s_old * corr + jnp.sum(ew)
    pav = lax.dot_general(ew, fv_ref[...], (((1,), (0,)), ((), ())),
                          preferred_element_type=jnp.float32)
    accf_ref[...] = accf_ref[...] * corr + pav

    rkT = rkT_ref[0]
    ksq_r = rkT * rkT + EPS
    sims_r = (rkT * qr) / (jnp.sqrt(ksq_r) * jnp.sqrt(qnsq_r))

    m_old_r = st_ref[2]
    s_old_r = st_ref[3]
    m_new_r = jnp.maximum(m_old_r, jnp.max(sims_r))
    corr_r = jnp.exp(m_old_r - m_new_r)
    ew_r = jnp.exp(sims_r - m_new_r)
    st_ref[2] = m_new_r
    st_ref[3] = s_old_r * corr_r + jnp.sum(ew_r)
    pav_r = lax.dot_general(ew_r, rv_ref[...], (((1,), (0,)), ((), ())),
                            preferred_element_type=jnp.float32)
    accr_ref[...] = accr_ref[...] * corr_r + pav_r

    @pl.when(i == NBLK - 1)
    def _finish():
        memfun = jnp.tanh(accf_ref[...] / st_ref[1])
        memrule = jnp.tanh(accr_ref[...] / st_ref[3])

        h2 = h_ref[...]
        c2 = c_ref[...]
        preact = (lax.dot_general(xt, wi_ref[...], (((1,), (0,)), ((), ())),
                                  preferred_element_type=jnp.float32)
                  + bi_ref[...]
                  + lax.dot_general(h2, wh_ref[...], (((1,), (0,)), ((), ())),
                                    preferred_element_type=jnp.float32)
                  + bh_ref[...])
        g = jax.nn.sigmoid(preact[:, :NG * HID])
        f_t = g[:, :HID]
        i_t = g[:, HID:2 * HID]
        o_t = g[:, 2 * HID:3 * HID]
        fun_t = g[:, 3 * HID:4 * HID]
        rul_t = g[:, 4 * HID:5 * HID]
        c_new = jnp.tanh(preact[:, NG * HID:])
        c_t = f_t * c2 + i_t * c_new + fun_t * memfun + rul_t * memrule
        h_t = o_t * jnp.tanh(c_t)

        sh = jnp.maximum(
            lax.dot_general(h_t, ws_ref[...], (((1,), (0,)), ((), ())),
                            preferred_element_type=jnp.float32) + bs_ref[...],
            0.0)
        logits = lax.dot_general(sh, wa_ref[...], (((1,), (0,)), ((), ())),
                                 preferred_element_type=jnp.float32) + ba_ref[...]
        lmax = jnp.max(logits)
        el = jnp.exp(logits - lmax)
        pi = el / jnp.sum(el)
        v = (jnp.sum(sh * wv_ref[...], axis=1, keepdims=True)
             + bv_ref[...])

        ht_out[...] = h_t
        ct_out[...] = c_t
        pi_out[...] = pi
        v_out[...] = v


def kernel(x_t, h, c, W_i2h, b_i2h, W_h2h, b_h2h, f_keys, f_vals,
           r_keys, r_vals, W_s, b_s, W_a, b_a, W_v, b_v):
    xt = x_t.reshape(1, IN_DIM)
    h2 = h.reshape(1, HID)
    c2 = c.reshape(1, HID)
    fk3 = f_keys.reshape(NBLK, BLK, FDIM).transpose(0, 2, 1)
    rk3 = r_keys.reshape(NBLK, 1, BLK)
    bi = b_i2h.reshape(1, -1)
    bh = b_h2h.reshape(1, -1)
    bs = b_s.reshape(1, -1)
    ba = b_a.reshape(1, -1)
    bv = b_v.reshape(1, 1)
    wvT = W_v.reshape(1, HID)

    cidx = lambda i: (0, 0)
    h_t, c_t, pi, v = pl.pallas_call(
        _body,
        grid=(NBLK,),
        in_specs=[
            pl.BlockSpec((1, IN_DIM), cidx),
            pl.BlockSpec((1, HID), cidx),
            pl.BlockSpec((1, HID), cidx),
            pl.BlockSpec((IN_DIM, (NG + 1) * HID), cidx),
            pl.BlockSpec((1, (NG + 1) * HID), cidx),
            pl.BlockSpec((HID, (NG + 1) * HID), cidx),
            pl.BlockSpec((1, (NG + 1) * HID), cidx),
            pl.BlockSpec((1, FDIM, BLK), lambda i: (i, 0, 0)),
            pl.BlockSpec((BLK, HID), lambda i: (i, 0)),
            pl.BlockSpec((1, 1, BLK), lambda i: (i, 0, 0)),
            pl.BlockSpec((BLK, HID), lambda i: (i, 0)),
            pl.BlockSpec((HID, HID), cidx),
            pl.BlockSpec((1, HID), cidx),
            pl.BlockSpec((HID, OUT), cidx),
            pl.BlockSpec((1, OUT), cidx),
            pl.BlockSpec((1, HID), cidx),
            pl.BlockSpec((1, 1), cidx),
        ],
        out_specs=[
            pl.BlockSpec((1, HID), cidx),
            pl.BlockSpec((1, HID), cidx),
            pl.BlockSpec((1, OUT), cidx),
            pl.BlockSpec((1, 1), cidx),
        ],
        out_shape=[
            jax.ShapeDtypeStruct((1, HID), jnp.float32),
            jax.ShapeDtypeStruct((1, HID), jnp.float32),
            jax.ShapeDtypeStruct((1, OUT), jnp.float32),
            jax.ShapeDtypeStruct((1, 1), jnp.float32),
        ],
        scratch_shapes=[
            pltpu.SMEM((4,), jnp.float32),
            pltpu.VMEM((1, HID), jnp.float32),
            pltpu.VMEM((1, HID), jnp.float32),
        ],
        compiler_params=pltpu.CompilerParams(
            dimension_semantics=("arbitrary",)),
    )(xt, h2, c2, W_i2h, bi, W_h2h, bh, fk3, f_vals, rk3, r_vals,
      W_s, bs, W_a, ba, wvT, bv)

    a_t = jax.random.categorical(jax.random.key(1), jnp.log(pi + EPS), axis=-1)
    prob_a_t = jnp.log(pi[0, a_t[0]] + EPS)
    h_out = h_t.reshape(1, 1, HID)
    c_out = c_t.reshape(1, 1, HID)
    return (a_t, prob_a_t, v, h_out, c_out, pi)
